# Initial kernel scaffold; baseline (speedup 1.0000x reference)
import jax
import jax.numpy as jnp
from jax.experimental import pallas as pl


def kernel(z, pos, edge_index, emb, msg_w1, msg_b1, msg_w2, msg_b2, upd_w1, upd_b1, upd_w2, upd_b2, out_w1, out_b1, out_w2, out_b2):
    raise NotImplementedError("write your pallas kernel here")


# speedup vs baseline: 3.2923x; 3.2923x over previous
import functools

import jax
import jax.numpy as jnp
from jax import lax
from jax.experimental import pallas as pl
from jax.experimental.pallas import tpu as pltpu
from jax.experimental.pallas import tpu_sc as plsc

N = 10000
E = 320000
HID = 128
RBF = 32
NL = 4
NUM_TYPES = 11
CUTOFF = 10.0
GAMMA = 1.0 / (CUTOFF / RBF)

NPAD = 10240
NC = 2
NS = 16
NW = NC * NS
EW = E // NW
CH = 80
NCHUNK = EW // CH
ROWS_PER_SUB = NPAD // NS
ROWS_N_SUB = N // NS
D2CH = 2000
ND2 = EW // D2CH

_mesh = functools.partial(
    plsc.VectorSubcoreMesh, core_axis_name="c", subcore_axis_name="s")


def _sc_geom_body(ii, jj, posp, e1, z16, sq_out, degp_out,
                  ii_all, jj_all, ii80, av, bv, e1v, sa, sb, deg_sh):
    cid = lax.axis_index("c")
    sid = lax.axis_index("s")
    w = cid * NS + sid

    pltpu.sync_copy(e1, e1v)
    pltpu.sync_copy(z16.at[pl.ds(sid * ROWS_PER_SUB, ROWS_PER_SUB)],
                    deg_sh.at[pl.ds(sid * ROWS_PER_SUB, ROWS_PER_SUB)])
    pltpu.sync_copy(ii.at[pl.ds(w * EW, EW)], ii_all)
    pltpu.sync_copy(jj.at[pl.ds(w * EW, EW)], jj_all)
    plsc.subcore_barrier()

    def chunk(n, _):
        off = n * CH
        base = w * EW + off
        ca = pltpu.async_copy(posp.at[ii_all.at[pl.ds(off, CH)]], av, sa)
        cb = pltpu.async_copy(posp.at[jj_all.at[pl.ds(off, CH)]], bv, sb)
        for q in range(CH // 16):
            s = pl.ds(q * 16, 16)
            ii80[s] = ii_all[pl.ds(off + q * 16, 16)]
        ca.wait()
        cb.wait()

        def crow(c, _):
            d = av[c, :] - bv[c, :]
            av[c, :] = d * d
            return 0

        lax.fori_loop(0, CH, crow, 0)
        pltpu.sync_copy(av, sq_out.at[pl.ds(base, CH)])
        pltpu.sync_copy(e1v, deg_sh.at[ii80], add=True)
        return 0

    lax.fori_loop(0, NCHUNK, chunk, 0)
    plsc.subcore_barrier()
    pltpu.sync_copy(deg_sh.at[pl.ds(sid * ROWS_PER_SUB, ROWS_PER_SUB)],
                    degp_out.at[cid, pl.ds(sid * ROWS_PER_SUB, ROWS_PER_SUB)])


def _sc_geom(idx_i, idx_j, posp, e1_rows, zeros16):
    return pl.kernel(
        _sc_geom_body,
        out_type=[jax.ShapeDtypeStruct((E, 16), jnp.float32),
                  jax.ShapeDtypeStruct((NC, NPAD, 16), jnp.float32)],
        mesh=_mesh(),
        compiler_params=pltpu.CompilerParams(use_tc_tiling_on_sc=False),
        scratch_types=[
            pltpu.VMEM((EW,), jnp.int32),
            pltpu.VMEM((EW,), jnp.int32),
            pltpu.VMEM((CH,), jnp.int32),
            pltpu.VMEM((CH, 16), jnp.float32),
            pltpu.VMEM((CH, 16), jnp.float32),
            pltpu.VMEM((CH, 16), jnp.float32),
            pltpu.SemaphoreType.DMA,
            pltpu.SemaphoreType.DMA,
            pltpu.VMEM_SHARED((NPAD, 16), jnp.float32),
        ],
    )(idx_i, idx_j, posp, e1_rows, zeros16)


def _sc_edge_body(ii, jj, pa, pb, rbfp, z128, accp_out,
                  ii_all, jj_all, ii80, av, bv, rv, sa, sb, sr, acc_sh):
    cid = lax.axis_index("c")
    sid = lax.axis_index("s")
    w = cid * NS + sid

    pltpu.sync_copy(z128.at[pl.ds(sid * ROWS_N_SUB, ROWS_N_SUB)],
                    acc_sh.at[pl.ds(sid * ROWS_N_SUB, ROWS_N_SUB)])
    pltpu.sync_copy(ii.at[pl.ds(w * EW, EW)], ii_all)
    pltpu.sync_copy(jj.at[pl.ds(w * EW, EW)], jj_all)
    plsc.subcore_barrier()

    def chunk(n, _):
        off = n * CH
        base = w * EW + off
        ca = pltpu.async_copy(pa.at[ii_all.at[pl.ds(off, CH)]], av, sa)
        cb = pltpu.async_copy(pb.at[jj_all.at[pl.ds(off, CH)]], bv, sb)
        cr = pltpu.async_copy(rbfp.at[pl.ds(base, CH)], rv, sr)
        for q in range(CH // 16):
            s = pl.ds(q * 16, 16)
            ii80[s] = ii_all[pl.ds(off + q * 16, 16)]
        ca.wait()
        cb.wait()
        cr.wait()

        def crow(c, _):
            for q in range(HID // 16):
                s = pl.ds(q * 16, 16)
                av[c, s] = jnp.maximum(av[c, s] + bv[c, s] + rv[c, s], 0.0)
            return 0

        lax.fori_loop(0, CH, crow, 0)
        pltpu.sync_copy(av, acc_sh.at[ii80], add=True)
        return 0

    lax.fori_loop(0, NCHUNK, chunk, 0)
    plsc.subcore_barrier()
    pltpu.sync_copy(acc_sh.at[pl.ds(sid * ROWS_N_SUB, ROWS_N_SUB)],
                    accp_out.at[cid, pl.ds(sid * ROWS_N_SUB, ROWS_N_SUB)])


def _sc_edge(idx_i, idx_j, pa, pb, rbfp_l, zeros128):
    return pl.kernel(
        _sc_edge_body,
        out_type=jax.ShapeDtypeStruct((NC, NPAD, HID), jnp.float32),
        mesh=_mesh(),
        compiler_params=pltpu.CompilerParams(use_tc_tiling_on_sc=False),
        scratch_types=[
            pltpu.VMEM((EW,), jnp.int32),
            pltpu.VMEM((EW,), jnp.int32),
            pltpu.VMEM((CH,), jnp.int32),
            pltpu.VMEM((CH, HID), jnp.float32),
            pltpu.VMEM((CH, HID), jnp.float32),
            pltpu.VMEM((CH, HID), jnp.float32),
            pltpu.SemaphoreType.DMA,
            pltpu.SemaphoreType.DMA,
            pltpu.SemaphoreType.DMA,
            pltpu.VMEM_SHARED((N, HID), jnp.float32),
        ],
    )(idx_i, idx_j, pa, pb, rbfp_l, zeros128)


_EB = 512


def _rbf_body(sq_ref, w1c_ref, out_ref):
    d2 = jnp.sum(sq_ref[...], axis=1, keepdims=True)
    dist = jnp.sqrt(d2)
    centers = lax.broadcasted_iota(jnp.int32, (1, RBF), 1).astype(
        jnp.float32) * (CUTOFF / (RBF - 1))
    dlt = dist - centers
    rbf = jnp.exp(-GAMMA * dlt * dlt)
    for l in range(NL):
        out_ref[l] = jnp.dot(rbf, w1c_ref[l],
                             preferred_element_type=jnp.float32)


def _k_rbf(sq, w1c):
    return pl.pallas_call(
        _rbf_body,
        grid=(E // _EB,),
        in_specs=[
            pl.BlockSpec((_EB, 16), lambda i: (i, 0)),
            pl.BlockSpec((NL, RBF, HID), lambda i: (0, 0, 0)),
        ],
        out_specs=pl.BlockSpec((NL, _EB, HID), lambda i: (0, i, 0)),
        out_shape=jax.ShapeDtypeStruct((NL, E, HID), jnp.float32),
    )(sq, w1c)


_NB = 256


def _prep_body(zb_ref, degp_ref, emb_ref, w1a_ref, w1b_ref, b1_ref,
               h_ref, pa_ref, pb_ref, deg_ref, inv_ref):
    zb = zb_ref[...]
    h = jnp.zeros((_NB, HID), jnp.float32)
    for t in range(NUM_TYPES):
        h = h + jnp.where(zb == t, 1.0, 0.0) * emb_ref[t:t + 1, :]
    dp = degp_ref[...]
    deg16 = dp[0] + dp[1]
    degc = deg16[:, 0:1]
    deg = jnp.broadcast_to(degc, (_NB, HID))
    inv = jnp.broadcast_to(1.0 / jnp.maximum(degc, 1.0), (_NB, HID))
    h_ref[...] = h
    pa_ref[...] = jnp.dot(h, w1a_ref[...],
                          preferred_element_type=jnp.float32) + b1_ref[...]
    pb_ref[...] = jnp.dot(h, w1b_ref[...], preferred_element_type=jnp.float32)
    deg_ref[...] = deg
    inv_ref[...] = inv


def _k_prep(z_bc, degp, emb, w1a0, w1b0, b1_0):
    o = jax.ShapeDtypeStruct((NPAD, HID), jnp.float32)
    return pl.pallas_call(
        _prep_body,
        grid=(NPAD // _NB,),
        in_specs=[
            pl.BlockSpec((_NB, HID), lambda i: (i, 0)),
            pl.BlockSpec((NC, _NB, 16), lambda i: (0, i, 0)),
            pl.BlockSpec((NUM_TYPES, HID), lambda i: (0, 0)),
            pl.BlockSpec((HID, HID), lambda i: (0, 0)),
            pl.BlockSpec((HID, HID), lambda i: (0, 0)),
            pl.BlockSpec((1, HID), lambda i: (0, 0)),
        ],
        out_specs=[pl.BlockSpec((_NB, HID), lambda i: (i, 0))] * 5,
        out_shape=[o, o, o, o, o],
    )(z_bc, degp, emb, w1a0, w1b0, b1_0)


def _post_body(want_next, accp_ref, deg_ref, inv_ref, h_ref,
               w2_ref, b2_ref, u1a_ref, u1b_ref, ub1_ref, u2_ref, ub2_ref,
               *rest):
    if want_next:
        wna_ref, wnb_ref, nb1_ref, hn_ref, pa_ref, pb_ref = rest
    else:
        (hn_ref,) = rest
    ap = accp_ref[...]
    acc = ap[0] + ap[1]
    aggr = (jnp.dot(acc, w2_ref[...], preferred_element_type=jnp.float32)
            + deg_ref[...] * b2_ref[...]) * inv_ref[...]
    h = h_ref[...]
    t = jnp.maximum(
        jnp.dot(h, u1a_ref[...], preferred_element_type=jnp.float32)
        + jnp.dot(aggr, u1b_ref[...], preferred_element_type=jnp.float32)
        + ub1_ref[...], 0.0)
    hn = jnp.dot(t, u2_ref[...], preferred_element_type=jnp.float32) + ub2_ref[...]
    hn_ref[...] = hn
    if want_next:
        pa_ref[...] = jnp.dot(hn, wna_ref[...],
                              preferred_element_type=jnp.float32) + nb1_ref[...]
        pb_ref[...] = jnp.dot(hn, wnb_ref[...],
                              preferred_element_type=jnp.float32)


def _k_post(accp, deg_b, inv_b, h, w2, b2, u1a, u1b, ub1, u2, ub2,
            nxt=None):
    want_next = nxt is not None
    o = jax.ShapeDtypeStruct((NPAD, HID), jnp.float32)
    full = lambda *shape: pl.BlockSpec(shape, lambda i: (0,) * len(shape))
    blk = pl.BlockSpec((_NB, HID), lambda i: (i, 0))
    in_specs = [
        pl.BlockSpec((NC, _NB, HID), lambda i: (0, i, 0)),
        blk, blk, blk,
        full(HID, HID), full(1, HID), full(HID, HID), full(HID, HID),
        full(1, HID), full(HID, HID), full(1, HID),
    ]
    args = [accp, deg_b, inv_b, h, w2, b2, u1a, u1b, ub1, u2, ub2]
    if want_next:
        in_specs += [full(HID, HID), full(HID, HID), full(1, HID)]
        args += list(nxt)
        out_specs, out_shape = [blk, blk, blk], [o, o, o]
    else:
        out_specs, out_shape = [blk], [o]
    return pl.pallas_call(
        functools.partial(_post_body, want_next),
        grid=(NPAD // _NB,),
        in_specs=in_specs,
        out_specs=out_specs,
        out_shape=out_shape,
    )(*args)


def _final_body(h_ref, ow1_ref, ob1_ref, ow2_ref, ob2_ref, out_ref):
    i = pl.program_id(0)
    h = h_ref[...]
    t = jnp.maximum(
        jnp.dot(h, ow1_ref[...], preferred_element_type=jnp.float32)
        + ob1_ref[...], 0.0)
    e = jnp.dot(t, ow2_ref[...], preferred_element_type=jnp.float32)
    row = lax.broadcasted_iota(jnp.int32, (_NB, 1), 0) + i * _NB
    s = jnp.sum(jnp.where(row < N, e, 0.0))
    sv = jnp.full((1, HID), s, jnp.float32)

    @pl.when(i == 0)
    def _():
        out_ref[...] = sv + N * ob2_ref[...]

    @pl.when(i != 0)
    def _():
        out_ref[...] += sv


def _k_final(h, ow1, ob1, ow2, ob2):
    return pl.pallas_call(
        _final_body,
        grid=(NPAD // _NB,),
        in_specs=[
            pl.BlockSpec((_NB, HID), lambda i: (i, 0)),
            pl.BlockSpec((HID, HID), lambda i: (0, 0)),
            pl.BlockSpec((1, HID), lambda i: (0, 0)),
            pl.BlockSpec((HID, 1), lambda i: (0, 0)),
            pl.BlockSpec((1, HID), lambda i: (0, 0)),
        ],
        out_specs=pl.BlockSpec((1, HID), lambda i: (0, 0)),
        out_shape=jax.ShapeDtypeStruct((1, HID), jnp.float32),
    )(h, ow1, ob1, ow2, ob2)


def kernel(z, pos, edge_index, emb, msg_w1, msg_b1, msg_w2, msg_b2,
           upd_w1, upd_b1, upd_w2, upd_b2, out_w1, out_b1, out_w2, out_b2):
    idx_i = edge_index[0].astype(jnp.int32)
    idx_j = edge_index[1].astype(jnp.int32)
    posp = jnp.zeros((N, 16), jnp.float32).at[:, :3].set(
        pos.astype(jnp.float32))

    e1_rows = jnp.zeros((CH, 16), jnp.float32).at[:, 0].set(1.0)
    zeros16 = jnp.zeros((NPAD, 16), jnp.float32)
    zeros128 = jnp.zeros((NPAD, HID), jnp.float32)

    sq, degp = _sc_geom(idx_i, idx_j, posp, e1_rows, zeros16)

    w1c = msg_w1[:, 2 * HID:, :]
    rbfp = _k_rbf(sq, w1c)

    z_pad = jnp.zeros((NPAD,), jnp.int32).at[:N].set(z.astype(jnp.int32))
    z_bc = jnp.broadcast_to(z_pad[:, None], (NPAD, HID))

    b1 = msg_b1.reshape(NL, 1, HID)
    b2 = msg_b2.reshape(NL, 1, HID)
    ub1 = upd_b1.reshape(NL, 1, HID)
    ub2 = upd_b2.reshape(NL, 1, HID)

    h, pa, pb, deg_b, inv_b = _k_prep(
        z_bc, degp, emb, msg_w1[0, :HID, :], msg_w1[0, HID:2 * HID, :], b1[0])

    for l in range(NL):
        accp = _sc_edge(idx_i, idx_j, pa, pb, rbfp[l], zeros128)
        if l < NL - 1:
            nxt = (msg_w1[l + 1, :HID, :], msg_w1[l + 1, HID:2 * HID, :],
                   b1[l + 1])
            h, pa, pb = _k_post(
                accp, deg_b, inv_b, h, msg_w2[l], b2[l],
                upd_w1[l, :HID, :], upd_w1[l, HID:, :], ub1[l],
                upd_w2[l], ub2[l], nxt=nxt)
        else:
            (h,) = _k_post(
                accp, deg_b, inv_b, h, msg_w2[l], b2[l],
                upd_w1[l, :HID, :], upd_w1[l, HID:, :], ub1[l],
                upd_w2[l], ub2[l])

    ob2_row = jnp.broadcast_to(out_b2.reshape(1, 1), (1, HID))
    res = _k_final(h, out_w1, out_b1.reshape(1, HID), out_w2, ob2_row)
    return res[0, 0]


# speedup vs baseline: 3.7686x; 1.1447x over previous
import functools

import jax
import jax.numpy as jnp
from jax import lax
from jax.experimental import pallas as pl
from jax.experimental.pallas import tpu as pltpu
from jax.experimental.pallas import tpu_sc as plsc

N = 10000
E = 320000
HID = 128
RBF = 32
NL = 4
NUM_TYPES = 11
CUTOFF = 10.0
GAMMA = 1.0 / (CUTOFF / RBF)

NPAD = 10240
NC = 2
NS = 16
NW = NC * NS
EW = E // NW
CH = 80
NCHUNK = EW // CH
ROWS_PER_SUB = NPAD // NS
ROWS_N_SUB = N // NS
D2CH = 2000
ND2 = EW // D2CH

_mesh = functools.partial(
    plsc.VectorSubcoreMesh, core_axis_name="c", subcore_axis_name="s")


def _sc_geom_body(ii, jj, posp, e1, z16, sq_out, degp_out,
                  ii_all, jj_all, ii80, av, bv, e1v, sa, sb, deg_sh):
    cid = lax.axis_index("c")
    sid = lax.axis_index("s")
    w = cid * NS + sid

    pltpu.sync_copy(e1, e1v)
    pltpu.sync_copy(z16.at[pl.ds(sid * ROWS_PER_SUB, ROWS_PER_SUB)],
                    deg_sh.at[pl.ds(sid * ROWS_PER_SUB, ROWS_PER_SUB)])
    pltpu.sync_copy(ii.at[pl.ds(w * EW, EW)], ii_all)
    pltpu.sync_copy(jj.at[pl.ds(w * EW, EW)], jj_all)
    plsc.subcore_barrier()

    def chunk(n, _):
        off = n * CH
        base = w * EW + off
        ca = pltpu.async_copy(posp.at[ii_all.at[pl.ds(off, CH)]], av, sa)
        cb = pltpu.async_copy(posp.at[jj_all.at[pl.ds(off, CH)]], bv, sb)
        for q in range(CH // 16):
            s = pl.ds(q * 16, 16)
            ii80[s] = ii_all[pl.ds(off + q * 16, 16)]
        ca.wait()
        cb.wait()

        def crow(c, _):
            d = av[c, :] - bv[c, :]
            av[c, :] = d * d
            return 0

        lax.fori_loop(0, CH, crow, 0)
        pltpu.sync_copy(av, sq_out.at[pl.ds(base, CH)])
        pltpu.sync_copy(e1v, deg_sh.at[ii80], add=True)
        return 0

    lax.fori_loop(0, NCHUNK, chunk, 0)
    plsc.subcore_barrier()
    pltpu.sync_copy(deg_sh.at[pl.ds(sid * ROWS_PER_SUB, ROWS_PER_SUB)],
                    degp_out.at[cid, pl.ds(sid * ROWS_PER_SUB, ROWS_PER_SUB)])


def _sc_geom(idx_i, idx_j, posp, e1_rows, zeros16):
    return pl.kernel(
        _sc_geom_body,
        out_type=[jax.ShapeDtypeStruct((E, 16), jnp.float32),
                  jax.ShapeDtypeStruct((NC, NPAD, 16), jnp.float32)],
        mesh=_mesh(),
        compiler_params=pltpu.CompilerParams(use_tc_tiling_on_sc=False),
        scratch_types=[
            pltpu.VMEM((EW,), jnp.int32),
            pltpu.VMEM((EW,), jnp.int32),
            pltpu.VMEM((CH,), jnp.int32),
            pltpu.VMEM((CH, 16), jnp.float32),
            pltpu.VMEM((CH, 16), jnp.float32),
            pltpu.VMEM((CH, 16), jnp.float32),
            pltpu.SemaphoreType.DMA,
            pltpu.SemaphoreType.DMA,
            pltpu.VMEM_SHARED((NPAD, 16), jnp.float32),
        ],
    )(idx_i, idx_j, posp, e1_rows, zeros16)


def _sc_edge_body(ii, jj, pa, pb, rbfp, z128, accp_out,
                  ii_all, jj_all, ii80, av, bv, rv, sa, sb, sr, acc_sh):
    cid = lax.axis_index("c")
    sid = lax.axis_index("s")
    w = cid * NS + sid

    pltpu.sync_copy(z128.at[pl.ds(sid * ROWS_N_SUB, ROWS_N_SUB)],
                    acc_sh.at[pl.ds(sid * ROWS_N_SUB, ROWS_N_SUB)])
    pltpu.sync_copy(ii.at[pl.ds(w * EW, EW)], ii_all)
    pltpu.sync_copy(jj.at[pl.ds(w * EW, EW)], jj_all)
    plsc.subcore_barrier()

    def chunk(n, _):
        off = n * CH
        base = w * EW + off
        ca = pltpu.async_copy(pa.at[ii_all.at[pl.ds(off, CH)]], av, sa)
        cb = pltpu.async_copy(pb.at[jj_all.at[pl.ds(off, CH)]], bv, sb)
        cr = pltpu.async_copy(rbfp.at[pl.ds(base, CH)], rv, sr)
        for q in range(CH // 16):
            s = pl.ds(q * 16, 16)
            ii80[s] = ii_all[pl.ds(off + q * 16, 16)]
        ca.wait()
        cb.wait()
        cr.wait()

        def crow(c, _):
            for q in range(HID // 16):
                s = pl.ds(q * 16, 16)
                av[c, s] = jnp.maximum(av[c, s] + bv[c, s] + rv[c, s], 0.0)
            return 0

        lax.fori_loop(0, CH, crow, 0)
        pltpu.sync_copy(av, acc_sh.at[ii80], add=True)
        return 0

    lax.fori_loop(0, NCHUNK, chunk, 0)
    plsc.subcore_barrier()
    pltpu.sync_copy(acc_sh.at[pl.ds(sid * ROWS_N_SUB, ROWS_N_SUB)],
                    accp_out.at[cid, pl.ds(sid * ROWS_N_SUB, ROWS_N_SUB)])


def _sc_edge(idx_i, idx_j, pa, pb, rbfp_l, zeros128):
    return pl.kernel(
        _sc_edge_body,
        out_type=jax.ShapeDtypeStruct((NC, NPAD, HID), jnp.float32),
        mesh=_mesh(),
        compiler_params=pltpu.CompilerParams(use_tc_tiling_on_sc=False),
        scratch_types=[
            pltpu.VMEM((EW,), jnp.int32),
            pltpu.VMEM((EW,), jnp.int32),
            pltpu.VMEM((CH,), jnp.int32),
            pltpu.VMEM((CH, HID), jnp.float32),
            pltpu.VMEM((CH, HID), jnp.float32),
            pltpu.VMEM((CH, HID), jnp.float32),
            pltpu.SemaphoreType.DMA,
            pltpu.SemaphoreType.DMA,
            pltpu.SemaphoreType.DMA,
            pltpu.VMEM_SHARED((N, HID), jnp.float32),
        ],
    )(idx_i, idx_j, pa, pb, rbfp_l, zeros128)


_EB = 512


def _rbf_body(sq_ref, w1c_ref, *out_ref):
    d2 = jnp.sum(sq_ref[...], axis=1, keepdims=True)
    dist = jnp.sqrt(d2)
    centers = lax.broadcasted_iota(jnp.int32, (1, RBF), 1).astype(
        jnp.float32) * (CUTOFF / (RBF - 1))
    dlt = dist - centers
    rbf = jnp.exp(-GAMMA * dlt * dlt)
    for l in range(NL):
        out_ref[l][...] = jnp.dot(rbf, w1c_ref[l],
                                  preferred_element_type=jnp.float32)


def _k_rbf(sq, w1c):
    return pl.pallas_call(
        _rbf_body,
        grid=(E // _EB,),
        in_specs=[
            pl.BlockSpec((_EB, 16), lambda i: (i, 0)),
            pl.BlockSpec((NL, RBF, HID), lambda i: (0, 0, 0)),
        ],
        out_specs=[pl.BlockSpec((_EB, HID), lambda i: (i, 0))] * NL,
        out_shape=[jax.ShapeDtypeStruct((E, HID), jnp.float32)] * NL,
    )(sq, w1c)


_NB = 256


def _prep_body(zb_ref, degp_ref, emb_ref, w1a_ref, w1b_ref, b1_ref,
               h_ref, pa_ref, pb_ref, deg_ref, inv_ref):
    zb = zb_ref[...]
    h = jnp.zeros((_NB, HID), jnp.float32)
    for t in range(NUM_TYPES):
        h = h + jnp.where(zb == t, 1.0, 0.0) * emb_ref[t:t + 1, :]
    dp = degp_ref[...]
    deg16 = dp[0] + dp[1]
    degc = deg16[:, 0:1]
    deg = jnp.broadcast_to(degc, (_NB, HID))
    inv = jnp.broadcast_to(1.0 / jnp.maximum(degc, 1.0), (_NB, HID))
    h_ref[...] = h
    pa_ref[...] = jnp.dot(h, w1a_ref[...],
                          preferred_element_type=jnp.float32) + b1_ref[...]
    pb_ref[...] = jnp.dot(h, w1b_ref[...], preferred_element_type=jnp.float32)
    deg_ref[...] = deg
    inv_ref[...] = inv


def _k_prep(z_bc, degp, emb, w1a0, w1b0, b1_0):
    o = jax.ShapeDtypeStruct((NPAD, HID), jnp.float32)
    return pl.pallas_call(
        _prep_body,
        grid=(NPAD // _NB,),
        in_specs=[
            pl.BlockSpec((_NB, HID), lambda i: (i, 0)),
            pl.BlockSpec((NC, _NB, 16), lambda i: (0, i, 0)),
            pl.BlockSpec((NUM_TYPES, HID), lambda i: (0, 0)),
            pl.BlockSpec((HID, HID), lambda i: (0, 0)),
            pl.BlockSpec((HID, HID), lambda i: (0, 0)),
            pl.BlockSpec((1, HID), lambda i: (0, 0)),
        ],
        out_specs=[pl.BlockSpec((_NB, HID), lambda i: (i, 0))] * 5,
        out_shape=[o, o, o, o, o],
    )(z_bc, degp, emb, w1a0, w1b0, b1_0)


def _post_body(want_next, accp_ref, deg_ref, inv_ref, h_ref,
               w2_ref, b2_ref, u1a_ref, u1b_ref, ub1_ref, u2_ref, ub2_ref,
               *rest):
    if want_next:
        wna_ref, wnb_ref, nb1_ref, hn_ref, pa_ref, pb_ref = rest
    else:
        (hn_ref,) = rest
    ap = accp_ref[...]
    acc = ap[0] + ap[1]
    aggr = (jnp.dot(acc, w2_ref[...], preferred_element_type=jnp.float32)
            + deg_ref[...] * b2_ref[...]) * inv_ref[...]
    h = h_ref[...]
    t = jnp.maximum(
        jnp.dot(h, u1a_ref[...], preferred_element_type=jnp.float32)
        + jnp.dot(aggr, u1b_ref[...], preferred_element_type=jnp.float32)
        + ub1_ref[...], 0.0)
    hn = jnp.dot(t, u2_ref[...], preferred_element_type=jnp.float32) + ub2_ref[...]
    hn_ref[...] = hn
    if want_next:
        pa_ref[...] = jnp.dot(hn, wna_ref[...],
                              preferred_element_type=jnp.float32) + nb1_ref[...]
        pb_ref[...] = jnp.dot(hn, wnb_ref[...],
                              preferred_element_type=jnp.float32)


def _k_post(accp, deg_b, inv_b, h, w2, b2, u1a, u1b, ub1, u2, ub2,
            nxt=None):
    want_next = nxt is not None
    o = jax.ShapeDtypeStruct((NPAD, HID), jnp.float32)
    full = lambda *shape: pl.BlockSpec(shape, lambda i: (0,) * len(shape))
    blk = pl.BlockSpec((_NB, HID), lambda i: (i, 0))
    in_specs = [
        pl.BlockSpec((NC, _NB, HID), lambda i: (0, i, 0)),
        blk, blk, blk,
        full(HID, HID), full(1, HID), full(HID, HID), full(HID, HID),
        full(1, HID), full(HID, HID), full(1, HID),
    ]
    args = [accp, deg_b, inv_b, h, w2, b2, u1a, u1b, ub1, u2, ub2]
    if want_next:
        in_specs += [full(HID, HID), full(HID, HID), full(1, HID)]
        args += list(nxt)
        out_specs, out_shape = [blk, blk, blk], [o, o, o]
    else:
        out_specs, out_shape = [blk], [o]
    return pl.pallas_call(
        functools.partial(_post_body, want_next),
        grid=(NPAD // _NB,),
        in_specs=in_specs,
        out_specs=out_specs,
        out_shape=out_shape,
    )(*args)


def _final_body(h_ref, ow1_ref, ob1_ref, ow2_ref, ob2_ref, out_ref):
    i = pl.program_id(0)
    h = h_ref[...]
    t = jnp.maximum(
        jnp.dot(h, ow1_ref[...], preferred_element_type=jnp.float32)
        + ob1_ref[...], 0.0)
    e = jnp.dot(t, ow2_ref[...], preferred_element_type=jnp.float32)
    row = lax.broadcasted_iota(jnp.int32, (_NB, 1), 0) + i * _NB
    s = jnp.sum(jnp.where(row < N, e, 0.0))
    sv = jnp.full((1, HID), s, jnp.float32)

    @pl.when(i == 0)
    def _():
        out_ref[...] = sv + N * ob2_ref[...]

    @pl.when(i != 0)
    def _():
        out_ref[...] += sv


def _k_final(h, ow1, ob1, ow2, ob2):
    return pl.pallas_call(
        _final_body,
        grid=(NPAD // _NB,),
        in_specs=[
            pl.BlockSpec((_NB, HID), lambda i: (i, 0)),
            pl.BlockSpec((HID, HID), lambda i: (0, 0)),
            pl.BlockSpec((1, HID), lambda i: (0, 0)),
            pl.BlockSpec((HID, 1), lambda i: (0, 0)),
            pl.BlockSpec((1, HID), lambda i: (0, 0)),
        ],
        out_specs=pl.BlockSpec((1, HID), lambda i: (0, 0)),
        out_shape=jax.ShapeDtypeStruct((1, HID), jnp.float32),
    )(h, ow1, ob1, ow2, ob2)


def kernel(z, pos, edge_index, emb, msg_w1, msg_b1, msg_w2, msg_b2,
           upd_w1, upd_b1, upd_w2, upd_b2, out_w1, out_b1, out_w2, out_b2):
    idx_i = edge_index[0].astype(jnp.int32)
    idx_j = edge_index[1].astype(jnp.int32)
    posp = jnp.pad(pos.astype(jnp.float32), ((0, 0), (0, 13)))

    e1_rows = jnp.zeros((CH, 16), jnp.float32).at[:, 0].set(1.0)
    zeros16 = jnp.zeros((NPAD, 16), jnp.float32)
    zeros128 = jnp.zeros((NPAD, HID), jnp.float32)

    sq, degp = _sc_geom(idx_i, idx_j, posp, e1_rows, zeros16)

    w1c = msg_w1[:, 2 * HID:, :]
    rbfp = _k_rbf(sq, w1c)

    z_pad = jnp.zeros((NPAD,), jnp.int32).at[:N].set(z.astype(jnp.int32))
    z_bc = jnp.broadcast_to(z_pad[:, None], (NPAD, HID))

    b1 = msg_b1.reshape(NL, 1, HID)
    b2 = msg_b2.reshape(NL, 1, HID)
    ub1 = upd_b1.reshape(NL, 1, HID)
    ub2 = upd_b2.reshape(NL, 1, HID)

    h, pa, pb, deg_b, inv_b = _k_prep(
        z_bc, degp, emb, msg_w1[0, :HID, :], msg_w1[0, HID:2 * HID, :], b1[0])

    for l in range(NL):
        accp = _sc_edge(idx_i, idx_j, pa, pb, rbfp[l], zeros128)
        if l < NL - 1:
            nxt = (msg_w1[l + 1, :HID, :], msg_w1[l + 1, HID:2 * HID, :],
                   b1[l + 1])
            h, pa, pb = _k_post(
                accp, deg_b, inv_b, h, msg_w2[l], b2[l],
                upd_w1[l, :HID, :], upd_w1[l, HID:, :], ub1[l],
                upd_w2[l], ub2[l], nxt=nxt)
        else:
            (h,) = _k_post(
                accp, deg_b, inv_b, h, msg_w2[l], b2[l],
                upd_w1[l, :HID, :], upd_w1[l, HID:, :], ub1[l],
                upd_w2[l], ub2[l])

    ob2_row = jnp.broadcast_to(out_b2.reshape(1, 1), (1, HID))
    res = _k_final(h, out_w1, out_b1.reshape(1, HID), out_w2, ob2_row)
    return res[0, 0]


# speedup vs baseline: 5.2801x; 1.4011x over previous
import functools

import jax
import jax.numpy as jnp
from jax import lax
from jax.experimental import pallas as pl
from jax.experimental.pallas import tpu as pltpu
from jax.experimental.pallas import tpu_sc as plsc

N = 10000
E = 320000
HID = 128
RBF = 32
NL = 4
NUM_TYPES = 11
CUTOFF = 10.0
GAMMA = 1.0 / (CUTOFF / RBF)

NPAD = 10240
NC = 2
NS = 16
NW = NC * NS
EW = E // NW
CH = 64
NCHUNK = EW // CH
CHT = EW - NCHUNK * CH
CHG = 80
NCHG = EW // CHG
ROWS_PER_SUB = NPAD // NS
ROWS_N_SUB = N // NS
D2CH = 2000
ND2 = EW // D2CH

_mesh = functools.partial(
    plsc.VectorSubcoreMesh, core_axis_name="c", subcore_axis_name="s")


def _sc_geom_body(ii, jj, posp, e1, z16, sq_out, degp_out,
                  ii_all, jj_all, ii80, av, bv, e1v, sa, sb, deg_sh):
    cid = lax.axis_index("c")
    sid = lax.axis_index("s")
    w = cid * NS + sid

    pltpu.sync_copy(e1, e1v)
    pltpu.sync_copy(z16.at[pl.ds(sid * ROWS_PER_SUB, ROWS_PER_SUB)],
                    deg_sh.at[pl.ds(sid * ROWS_PER_SUB, ROWS_PER_SUB)])
    pltpu.sync_copy(ii.at[pl.ds(w * EW, EW)], ii_all)
    pltpu.sync_copy(jj.at[pl.ds(w * EW, EW)], jj_all)
    plsc.subcore_barrier()

    def chunk(n, _):
        off = n * CHG
        base = w * EW + off
        ca = pltpu.async_copy(posp.at[ii_all.at[pl.ds(off, CHG)]], av, sa)
        cb = pltpu.async_copy(posp.at[jj_all.at[pl.ds(off, CHG)]], bv, sb)
        for q in range(CHG // 16):
            s = pl.ds(q * 16, 16)
            ii80[s] = ii_all[pl.ds(off + q * 16, 16)]
        ca.wait()
        cb.wait()

        def crow(c, _):
            d = av[c, :] - bv[c, :]
            av[c, :] = d * d
            return 0

        lax.fori_loop(0, CHG, crow, 0)
        pltpu.sync_copy(av, sq_out.at[pl.ds(base, CHG)])
        pltpu.sync_copy(e1v, deg_sh.at[ii80], add=True)
        return 0

    lax.fori_loop(0, NCHG, chunk, 0)
    plsc.subcore_barrier()
    pltpu.sync_copy(deg_sh.at[pl.ds(sid * ROWS_PER_SUB, ROWS_PER_SUB)],
                    degp_out.at[cid, pl.ds(sid * ROWS_PER_SUB, ROWS_PER_SUB)])


def _sc_geom(idx_i, idx_j, posp, e1_rows, zeros16):
    return pl.kernel(
        _sc_geom_body,
        out_type=[jax.ShapeDtypeStruct((E, 16), jnp.float32),
                  jax.ShapeDtypeStruct((NC, NPAD, 16), jnp.float32)],
        mesh=_mesh(),
        compiler_params=pltpu.CompilerParams(use_tc_tiling_on_sc=False),
        scratch_types=[
            pltpu.VMEM((EW,), jnp.int32),
            pltpu.VMEM((EW,), jnp.int32),
            pltpu.VMEM((CHG,), jnp.int32),
            pltpu.VMEM((CHG, 16), jnp.float32),
            pltpu.VMEM((CHG, 16), jnp.float32),
            pltpu.VMEM((CHG, 16), jnp.float32),
            pltpu.SemaphoreType.DMA,
            pltpu.SemaphoreType.DMA,
            pltpu.VMEM_SHARED((NPAD, 16), jnp.float32),
        ],
    )(idx_i, idx_j, posp, e1_rows, zeros16)


def _sc_edge_body(ii, jj, pa, pb, rbfp, z128, accp_out,
                  ii0, jj0, iiS0, av0, bv0, rv0,
                  ii1, jj1, iiS1, av1, bv1, rv1,
                  iit, jjt,
                  sa0, sb0, sr0, ss0, si0, sa1, sb1, sr1, ss1, si1,
                  acc_sh):
    cid = lax.axis_index("c")
    sid = lax.axis_index("s")
    w = cid * NS + sid
    base0 = w * EW
    bufs = ((ii0, jj0, iiS0, av0, bv0, rv0, sa0, sb0, sr0, ss0, si0),
            (ii1, jj1, iiS1, av1, bv1, rv1, sa1, sb1, sr1, ss1, si1))

    pltpu.sync_copy(z128.at[pl.ds(sid * ROWS_N_SUB, ROWS_N_SUB)],
                    acc_sh.at[pl.ds(sid * ROWS_N_SUB, ROWS_N_SUB)])
    plsc.subcore_barrier()

    def idx_start(n, b):
        iiv, jjv, iiS, av, bv, rv, sa, sb, sr, ss, si = bufs[b]
        pltpu.async_copy(ii.at[pl.ds(base0 + n * CH, CH)], iiv, si)
        pltpu.async_copy(jj.at[pl.ds(base0 + n * CH, CH)], jjv, si)

    def rv_start(n, b):
        iiv, jjv, iiS, av, bv, rv, sa, sb, sr, ss, si = bufs[b]
        pltpu.async_copy(rbfp.at[pl.ds(base0 + n * CH, CH)], rv, sr)

    def wait_scatter(b):
        iiv, jjv, iiS, av, bv, rv, sa, sb, sr, ss, si = bufs[b]
        pltpu.make_async_copy(av, acc_sh.at[iiS], ss).wait()

    def gather_start(n, b, first):
        iiv, jjv, iiS, av, bv, rv, sa, sb, sr, ss, si = bufs[b]
        if not first:
            wait_scatter(b)
        pltpu.make_async_copy(ii.at[pl.ds(base0 + n * CH, CH)], iiv, si).wait()
        pltpu.make_async_copy(jj.at[pl.ds(base0 + n * CH, CH)], jjv, si).wait()
        pltpu.async_copy(pa.at[iiv], av, sa)
        pltpu.async_copy(pb.at[jjv], bv, sb)

    def finish(n, b, prefetch=True):
        iiv, jjv, iiS, av, bv, rv, sa, sb, sr, ss, si = bufs[b]
        pltpu.make_async_copy(pa.at[iiv], av, sa).wait()
        pltpu.make_async_copy(pb.at[jjv], bv, sb).wait()
        pltpu.make_async_copy(rbfp.at[pl.ds(base0 + n * CH, CH)], rv, sr).wait()
        for q in range(CH // 16):
            s = pl.ds(q * 16, 16)
            iiS[s] = iiv[s]
        if prefetch:
            idx_start(n + 2, b)

        def crow(c, _):
            for q in range(HID // 16):
                s = pl.ds(q * 16, 16)
                av[c, s] = jnp.maximum(av[c, s] + bv[c, s] + rv[c, s], 0.0)
            return 0

        lax.fori_loop(0, CH, crow, 0)
        if prefetch:
            rv_start(n + 2, b)
        pltpu.async_copy(av, acc_sh.at[iiS], ss, add=True)

    idx_start(0, 0)
    idx_start(1, 1)
    rv_start(0, 0)
    rv_start(1, 1)
    gather_start(0, 0, True)
    gather_start(1, 1, True)

    def body(m, _):
        n0 = 2 * m
        finish(n0, 0)
        gather_start(n0 + 2, 0, False)
        finish(n0 + 1, 1)
        gather_start(n0 + 3, 1, False)
        return 0

    lax.fori_loop(0, NCHUNK // 2 - 1, body, 0)
    finish(NCHUNK - 2, 0, prefetch=False)
    finish(NCHUNK - 1, 1, prefetch=False)
    wait_scatter(0)
    wait_scatter(1)

    toff = base0 + NCHUNK * CH
    pltpu.sync_copy(ii.at[pl.ds(toff, CHT)], iit)
    pltpu.sync_copy(jj.at[pl.ds(toff, CHT)], jjt)
    pltpu.sync_copy(pa.at[iit], av0.at[pl.ds(0, CHT)])
    pltpu.sync_copy(pb.at[jjt], bv0.at[pl.ds(0, CHT)])
    pltpu.sync_copy(rbfp.at[pl.ds(toff, CHT)], rv0.at[pl.ds(0, CHT)])

    def trow(c, _):
        for q in range(HID // 16):
            s = pl.ds(q * 16, 16)
            av0[c, s] = jnp.maximum(av0[c, s] + bv0[c, s] + rv0[c, s], 0.0)
        return 0

    lax.fori_loop(0, CHT, trow, 0)
    pltpu.sync_copy(av0.at[pl.ds(0, CHT)], acc_sh.at[iit], add=True)

    plsc.subcore_barrier()
    pltpu.sync_copy(acc_sh.at[pl.ds(sid * ROWS_N_SUB, ROWS_N_SUB)],
                    accp_out.at[cid, pl.ds(sid * ROWS_N_SUB, ROWS_N_SUB)])


def _sc_edge(idx_i, idx_j, pa, pb, rbfp_l, zeros128):
    sets = [
        pltpu.VMEM((CH,), jnp.int32),
        pltpu.VMEM((CH,), jnp.int32),
        pltpu.VMEM((CH,), jnp.int32),
        pltpu.VMEM((CH, HID), jnp.float32),
        pltpu.VMEM((CH, HID), jnp.float32),
        pltpu.VMEM((CH, HID), jnp.float32),
    ]
    return pl.kernel(
        _sc_edge_body,
        out_type=jax.ShapeDtypeStruct((NC, NPAD, HID), jnp.float32),
        mesh=_mesh(),
        compiler_params=pltpu.CompilerParams(use_tc_tiling_on_sc=False),
        scratch_types=(
            sets + sets
            + [pltpu.VMEM((CHT,), jnp.int32), pltpu.VMEM((CHT,), jnp.int32)]
            + [pltpu.SemaphoreType.DMA] * 10
            + [pltpu.VMEM_SHARED((N, HID), jnp.float32)]
        ),
    )(idx_i, idx_j, pa, pb, rbfp_l, zeros128)


_EB = 512


def _rbf_body(sq_ref, w1c_ref, *out_ref):
    d2 = jnp.sum(sq_ref[...], axis=1, keepdims=True)
    dist = jnp.sqrt(d2)
    centers = lax.broadcasted_iota(jnp.int32, (1, RBF), 1).astype(
        jnp.float32) * (CUTOFF / (RBF - 1))
    dlt = dist - centers
    rbf = jnp.exp(-GAMMA * dlt * dlt)
    for l in range(NL):
        out_ref[l][...] = jnp.dot(rbf, w1c_ref[l],
                                  preferred_element_type=jnp.float32)


def _k_rbf(sq, w1c):
    return pl.pallas_call(
        _rbf_body,
        grid=(E // _EB,),
        in_specs=[
            pl.BlockSpec((_EB, 16), lambda i: (i, 0)),
            pl.BlockSpec((NL, RBF, HID), lambda i: (0, 0, 0)),
        ],
        out_specs=[pl.BlockSpec((_EB, HID), lambda i: (i, 0))] * NL,
        out_shape=[jax.ShapeDtypeStruct((E, HID), jnp.float32)] * NL,
    )(sq, w1c)


_NB = 256


def _prep_body(zb_ref, degp_ref, emb_ref, w1a_ref, w1b_ref, b1_ref,
               h_ref, pa_ref, pb_ref, deg_ref, inv_ref):
    zb = zb_ref[...]
    h = jnp.zeros((_NB, HID), jnp.float32)
    for t in range(NUM_TYPES):
        h = h + jnp.where(zb == t, 1.0, 0.0) * emb_ref[t:t + 1, :]
    dp = degp_ref[...]
    deg16 = dp[0] + dp[1]
    degc = deg16[:, 0:1]
    deg = jnp.broadcast_to(degc, (_NB, HID))
    inv = jnp.broadcast_to(1.0 / jnp.maximum(degc, 1.0), (_NB, HID))
    h_ref[...] = h
    pa_ref[...] = jnp.dot(h, w1a_ref[...],
                          preferred_element_type=jnp.float32) + b1_ref[...]
    pb_ref[...] = jnp.dot(h, w1b_ref[...], preferred_element_type=jnp.float32)
    deg_ref[...] = deg
    inv_ref[...] = inv


def _k_prep(z_bc, degp, emb, w1a0, w1b0, b1_0):
    o = jax.ShapeDtypeStruct((NPAD, HID), jnp.float32)
    return pl.pallas_call(
        _prep_body,
        grid=(NPAD // _NB,),
        in_specs=[
            pl.BlockSpec((_NB, HID), lambda i: (i, 0)),
            pl.BlockSpec((NC, _NB, 16), lambda i: (0, i, 0)),
            pl.BlockSpec((NUM_TYPES, HID), lambda i: (0, 0)),
            pl.BlockSpec((HID, HID), lambda i: (0, 0)),
            pl.BlockSpec((HID, HID), lambda i: (0, 0)),
            pl.BlockSpec((1, HID), lambda i: (0, 0)),
        ],
        out_specs=[pl.BlockSpec((_NB, HID), lambda i: (i, 0))] * 5,
        out_shape=[o, o, o, o, o],
    )(z_bc, degp, emb, w1a0, w1b0, b1_0)


def _post_body(want_next, accp_ref, deg_ref, inv_ref, h_ref,
               w2_ref, b2_ref, u1a_ref, u1b_ref, ub1_ref, u2_ref, ub2_ref,
               *rest):
    if want_next:
        wna_ref, wnb_ref, nb1_ref, hn_ref, pa_ref, pb_ref = rest
    else:
        (hn_ref,) = rest
    ap = accp_ref[...]
    acc = ap[0] + ap[1]
    aggr = (jnp.dot(acc, w2_ref[...], preferred_element_type=jnp.float32)
            + deg_ref[...] * b2_ref[...]) * inv_ref[...]
    h = h_ref[...]
    t = jnp.maximum(
        jnp.dot(h, u1a_ref[...], preferred_element_type=jnp.float32)
        + jnp.dot(aggr, u1b_ref[...], preferred_element_type=jnp.float32)
        + ub1_ref[...], 0.0)
    hn = jnp.dot(t, u2_ref[...], preferred_element_type=jnp.float32) + ub2_ref[...]
    hn_ref[...] = hn
    if want_next:
        pa_ref[...] = jnp.dot(hn, wna_ref[...],
                              preferred_element_type=jnp.float32) + nb1_ref[...]
        pb_ref[...] = jnp.dot(hn, wnb_ref[...],
                              preferred_element_type=jnp.float32)


def _k_post(accp, deg_b, inv_b, h, w2, b2, u1a, u1b, ub1, u2, ub2,
            nxt=None):
    want_next = nxt is not None
    o = jax.ShapeDtypeStruct((NPAD, HID), jnp.float32)
    full = lambda *shape: pl.BlockSpec(shape, lambda i: (0,) * len(shape))
    blk = pl.BlockSpec((_NB, HID), lambda i: (i, 0))
    in_specs = [
        pl.BlockSpec((NC, _NB, HID), lambda i: (0, i, 0)),
        blk, blk, blk,
        full(HID, HID), full(1, HID), full(HID, HID), full(HID, HID),
        full(1, HID), full(HID, HID), full(1, HID),
    ]
    args = [accp, deg_b, inv_b, h, w2, b2, u1a, u1b, ub1, u2, ub2]
    if want_next:
        in_specs += [full(HID, HID), full(HID, HID), full(1, HID)]
        args += list(nxt)
        out_specs, out_shape = [blk, blk, blk], [o, o, o]
    else:
        out_specs, out_shape = [blk], [o]
    return pl.pallas_call(
        functools.partial(_post_body, want_next),
        grid=(NPAD // _NB,),
        in_specs=in_specs,
        out_specs=out_specs,
        out_shape=out_shape,
    )(*args)


def _final_body(h_ref, ow1_ref, ob1_ref, ow2_ref, ob2_ref, out_ref):
    i = pl.program_id(0)
    h = h_ref[...]
    t = jnp.maximum(
        jnp.dot(h, ow1_ref[...], preferred_element_type=jnp.float32)
        + ob1_ref[...], 0.0)
    e = jnp.dot(t, ow2_ref[...], preferred_element_type=jnp.float32)
    row = lax.broadcasted_iota(jnp.int32, (_NB, 1), 0) + i * _NB
    s = jnp.sum(jnp.where(row < N, e, 0.0))
    sv = jnp.full((1, HID), s, jnp.float32)

    @pl.when(i == 0)
    def _():
        out_ref[...] = sv + N * ob2_ref[...]

    @pl.when(i != 0)
    def _():
        out_ref[...] += sv


def _k_final(h, ow1, ob1, ow2, ob2):
    return pl.pallas_call(
        _final_body,
        grid=(NPAD // _NB,),
        in_specs=[
            pl.BlockSpec((_NB, HID), lambda i: (i, 0)),
            pl.BlockSpec((HID, HID), lambda i: (0, 0)),
            pl.BlockSpec((1, HID), lambda i: (0, 0)),
            pl.BlockSpec((HID, 1), lambda i: (0, 0)),
            pl.BlockSpec((1, HID), lambda i: (0, 0)),
        ],
        out_specs=pl.BlockSpec((1, HID), lambda i: (0, 0)),
        out_shape=jax.ShapeDtypeStruct((1, HID), jnp.float32),
    )(h, ow1, ob1, ow2, ob2)


def kernel(z, pos, edge_index, emb, msg_w1, msg_b1, msg_w2, msg_b2,
           upd_w1, upd_b1, upd_w2, upd_b2, out_w1, out_b1, out_w2, out_b2):
    idx_i = edge_index[0].astype(jnp.int32)
    idx_j = edge_index[1].astype(jnp.int32)
    posp = jnp.pad(pos.astype(jnp.float32), ((0, 0), (0, 13)))

    e1_rows = jnp.zeros((CHG, 16), jnp.float32).at[:, 0].set(1.0)
    zeros16 = jnp.zeros((NPAD, 16), jnp.float32)
    zeros128 = jnp.zeros((NPAD, HID), jnp.float32)

    sq, degp = _sc_geom(idx_i, idx_j, posp, e1_rows, zeros16)

    w1c = msg_w1[:, 2 * HID:, :]
    rbfp = _k_rbf(sq, w1c)

    z_pad = jnp.zeros((NPAD,), jnp.int32).at[:N].set(z.astype(jnp.int32))
    z_bc = jnp.broadcast_to(z_pad[:, None], (NPAD, HID))

    b1 = msg_b1.reshape(NL, 1, HID)
    b2 = msg_b2.reshape(NL, 1, HID)
    ub1 = upd_b1.reshape(NL, 1, HID)
    ub2 = upd_b2.reshape(NL, 1, HID)

    h, pa, pb, deg_b, inv_b = _k_prep(
        z_bc, degp, emb, msg_w1[0, :HID, :], msg_w1[0, HID:2 * HID, :], b1[0])

    for l in range(NL):
        accp = _sc_edge(idx_i, idx_j, pa, pb, rbfp[l], zeros128)
        if l < NL - 1:
            nxt = (msg_w1[l + 1, :HID, :], msg_w1[l + 1, HID:2 * HID, :],
                   b1[l + 1])
            h, pa, pb = _k_post(
                accp, deg_b, inv_b, h, msg_w2[l], b2[l],
                upd_w1[l, :HID, :], upd_w1[l, HID:, :], ub1[l],
                upd_w2[l], ub2[l], nxt=nxt)
        else:
            (h,) = _k_post(
                accp, deg_b, inv_b, h, msg_w2[l], b2[l],
                upd_w1[l, :HID, :], upd_w1[l, HID:, :], ub1[l],
                upd_w2[l], ub2[l])

    ob2_row = jnp.broadcast_to(out_b2.reshape(1, 1), (1, HID))
    res = _k_final(h, out_w1, out_b1.reshape(1, HID), out_w2, ob2_row)
    return res[0, 0]


# speedup vs baseline: 5.6139x; 1.0632x over previous
import functools

import jax
import jax.numpy as jnp
from jax import lax
from jax.experimental import pallas as pl
from jax.experimental.pallas import tpu as pltpu
from jax.experimental.pallas import tpu_sc as plsc

N = 10000
E = 320000
HID = 128
RBF = 32
NL = 4
NUM_TYPES = 11
CUTOFF = 10.0
GAMMA = 1.0 / (CUTOFF / RBF)

NPAD = 10240
NC = 2
NS = 16
NW = NC * NS
EW = E // NW
CH = 64
NCHUNK = EW // CH
CHT = EW - NCHUNK * CH
CHG = 80
NCHG = EW // CHG
ROWS_PER_SUB = NPAD // NS
ROWS_N_SUB = N // NS
D2CH = 2000
ND2 = EW // D2CH

_mesh = functools.partial(
    plsc.VectorSubcoreMesh, core_axis_name="c", subcore_axis_name="s")


def _sc_geom_body(ii, jj, posp, zp, emb, e1, z16, sq_out, degp_out, h0_out,
                  ii_all, jj_all, ii80, av, bv, e1v, zc, hv, sa, sb, deg_sh):
    cid = lax.axis_index("c")
    sid = lax.axis_index("s")
    w = cid * NS + sid

    @pl.when(cid == 0)
    def _():
        for k in range(ROWS_PER_SUB // 128):
            nb = sid * ROWS_PER_SUB + k * 128
            pltpu.sync_copy(zp.at[pl.ds(nb, 128)], zc)
            pltpu.sync_copy(emb.at[zc], hv)
            pltpu.sync_copy(hv, h0_out.at[pl.ds(nb, 128)])

    pltpu.sync_copy(e1, e1v)
    pltpu.sync_copy(z16.at[pl.ds(sid * ROWS_PER_SUB, ROWS_PER_SUB)],
                    deg_sh.at[pl.ds(sid * ROWS_PER_SUB, ROWS_PER_SUB)])
    pltpu.sync_copy(ii.at[pl.ds(w * EW, EW)], ii_all)
    pltpu.sync_copy(jj.at[pl.ds(w * EW, EW)], jj_all)
    plsc.subcore_barrier()

    def chunk(n, _):
        off = n * CHG
        base = w * EW + off
        ca = pltpu.async_copy(posp.at[ii_all.at[pl.ds(off, CHG)]], av, sa)
        cb = pltpu.async_copy(posp.at[jj_all.at[pl.ds(off, CHG)]], bv, sb)
        for q in range(CHG // 16):
            s = pl.ds(q * 16, 16)
            ii80[s] = ii_all[pl.ds(off + q * 16, 16)]
        ca.wait()
        cb.wait()

        def crow(c, _):
            d = av[c, :] - bv[c, :]
            av[c, :] = d * d
            return 0

        lax.fori_loop(0, CHG, crow, 0)
        pltpu.sync_copy(av, sq_out.at[pl.ds(base, CHG)])
        pltpu.sync_copy(e1v, deg_sh.at[ii80], add=True)
        return 0

    lax.fori_loop(0, NCHG, chunk, 0)
    plsc.subcore_barrier()
    pltpu.sync_copy(deg_sh.at[pl.ds(sid * ROWS_PER_SUB, ROWS_PER_SUB)],
                    degp_out.at[cid, pl.ds(sid * ROWS_PER_SUB, ROWS_PER_SUB)])


def _sc_geom(idx_i, idx_j, posp, z_pad, emb, e1_rows, zeros16):
    return pl.kernel(
        _sc_geom_body,
        out_type=[jax.ShapeDtypeStruct((E, 16), jnp.float32),
                  jax.ShapeDtypeStruct((NC, NPAD, 16), jnp.float32),
                  jax.ShapeDtypeStruct((NPAD, HID), jnp.float32)],
        mesh=_mesh(),
        compiler_params=pltpu.CompilerParams(use_tc_tiling_on_sc=False),
        scratch_types=[
            pltpu.VMEM((EW,), jnp.int32),
            pltpu.VMEM((EW,), jnp.int32),
            pltpu.VMEM((CHG,), jnp.int32),
            pltpu.VMEM((CHG, 16), jnp.float32),
            pltpu.VMEM((CHG, 16), jnp.float32),
            pltpu.VMEM((CHG, 16), jnp.float32),
            pltpu.VMEM((128,), jnp.int32),
            pltpu.VMEM((128, HID), jnp.float32),
            pltpu.SemaphoreType.DMA,
            pltpu.SemaphoreType.DMA,
            pltpu.VMEM_SHARED((NPAD, 16), jnp.float32),
        ],
    )(idx_i, idx_j, posp, z_pad, emb, e1_rows, zeros16)


def _sc_edge_body(ii, jj, pa, pb, rbfp, z128, accp_out,
                  ii0, jj0, iiS0, av0, bv0, rv0,
                  ii1, jj1, iiS1, av1, bv1, rv1,
                  iit, jjt,
                  sa0, sb0, sr0, ss0, si0, sa1, sb1, sr1, ss1, si1,
                  acc_sh):
    cid = lax.axis_index("c")
    sid = lax.axis_index("s")
    w = cid * NS + sid
    base0 = w * EW
    bufs = ((ii0, jj0, iiS0, av0, bv0, rv0, sa0, sb0, sr0, ss0, si0),
            (ii1, jj1, iiS1, av1, bv1, rv1, sa1, sb1, sr1, ss1, si1))

    pltpu.sync_copy(z128.at[pl.ds(sid * ROWS_N_SUB, ROWS_N_SUB)],
                    acc_sh.at[pl.ds(sid * ROWS_N_SUB, ROWS_N_SUB)])
    plsc.subcore_barrier()

    def idx_start(n, b):
        iiv, jjv, iiS, av, bv, rv, sa, sb, sr, ss, si = bufs[b]
        pltpu.async_copy(ii.at[pl.ds(base0 + n * CH, CH)], iiv, si)
        pltpu.async_copy(jj.at[pl.ds(base0 + n * CH, CH)], jjv, si)

    def rv_start(n, b):
        iiv, jjv, iiS, av, bv, rv, sa, sb, sr, ss, si = bufs[b]
        pltpu.async_copy(rbfp.at[pl.ds(base0 + n * CH, CH)], rv, sr)

    def wait_scatter(b):
        iiv, jjv, iiS, av, bv, rv, sa, sb, sr, ss, si = bufs[b]
        pltpu.make_async_copy(av, acc_sh.at[iiS], ss).wait()

    def gather_start(n, b, first):
        iiv, jjv, iiS, av, bv, rv, sa, sb, sr, ss, si = bufs[b]
        if not first:
            wait_scatter(b)
        pltpu.make_async_copy(ii.at[pl.ds(base0 + n * CH, CH)], iiv, si).wait()
        pltpu.make_async_copy(jj.at[pl.ds(base0 + n * CH, CH)], jjv, si).wait()
        pltpu.async_copy(pa.at[iiv], av, sa)
        pltpu.async_copy(pb.at[jjv], bv, sb)

    def finish(n, b, prefetch=True):
        iiv, jjv, iiS, av, bv, rv, sa, sb, sr, ss, si = bufs[b]
        pltpu.make_async_copy(pa.at[iiv], av, sa).wait()
        pltpu.make_async_copy(pb.at[jjv], bv, sb).wait()
        pltpu.make_async_copy(rbfp.at[pl.ds(base0 + n * CH, CH)], rv, sr).wait()
        for q in range(CH // 16):
            s = pl.ds(q * 16, 16)
            iiS[s] = iiv[s]
        if prefetch:
            idx_start(n + 2, b)

        def crow(c, _):
            for q in range(HID // 16):
                s = pl.ds(q * 16, 16)
                av[c, s] = jnp.maximum(av[c, s] + bv[c, s] + rv[c, s], 0.0)
            return 0

        lax.fori_loop(0, CH, crow, 0)
        if prefetch:
            rv_start(n + 2, b)
        pltpu.async_copy(av, acc_sh.at[iiS], ss, add=True)

    idx_start(0, 0)
    idx_start(1, 1)
    rv_start(0, 0)
    rv_start(1, 1)
    gather_start(0, 0, True)
    gather_start(1, 1, True)

    def body(m, _):
        n0 = 2 * m
        finish(n0, 0)
        gather_start(n0 + 2, 0, False)
        finish(n0 + 1, 1)
        gather_start(n0 + 3, 1, False)
        return 0

    lax.fori_loop(0, NCHUNK // 2 - 1, body, 0)
    finish(NCHUNK - 2, 0, prefetch=False)
    finish(NCHUNK - 1, 1, prefetch=False)
    wait_scatter(0)
    wait_scatter(1)

    toff = base0 + NCHUNK * CH
    pltpu.sync_copy(ii.at[pl.ds(toff, CHT)], iit)
    pltpu.sync_copy(jj.at[pl.ds(toff, CHT)], jjt)
    pltpu.sync_copy(pa.at[iit], av0.at[pl.ds(0, CHT)])
    pltpu.sync_copy(pb.at[jjt], bv0.at[pl.ds(0, CHT)])
    pltpu.sync_copy(rbfp.at[pl.ds(toff, CHT)], rv0.at[pl.ds(0, CHT)])

    def trow(c, _):
        for q in range(HID // 16):
            s = pl.ds(q * 16, 16)
            av0[c, s] = jnp.maximum(av0[c, s] + bv0[c, s] + rv0[c, s], 0.0)
        return 0

    lax.fori_loop(0, CHT, trow, 0)
    pltpu.sync_copy(av0.at[pl.ds(0, CHT)], acc_sh.at[iit], add=True)

    plsc.subcore_barrier()
    pltpu.sync_copy(acc_sh.at[pl.ds(sid * ROWS_N_SUB, ROWS_N_SUB)],
                    accp_out.at[cid, pl.ds(sid * ROWS_N_SUB, ROWS_N_SUB)])


def _sc_edge(idx_i, idx_j, pa, pb, rbfp_l, zeros128):
    sets = [
        pltpu.VMEM((CH,), jnp.int32),
        pltpu.VMEM((CH,), jnp.int32),
        pltpu.VMEM((CH,), jnp.int32),
        pltpu.VMEM((CH, HID), jnp.float32),
        pltpu.VMEM((CH, HID), jnp.float32),
        pltpu.VMEM((CH, HID), jnp.float32),
    ]
    return pl.kernel(
        _sc_edge_body,
        out_type=jax.ShapeDtypeStruct((NC, NPAD, HID), jnp.float32),
        mesh=_mesh(),
        compiler_params=pltpu.CompilerParams(use_tc_tiling_on_sc=False),
        scratch_types=(
            sets + sets
            + [pltpu.VMEM((CHT,), jnp.int32), pltpu.VMEM((CHT,), jnp.int32)]
            + [pltpu.SemaphoreType.DMA] * 10
            + [pltpu.VMEM_SHARED((N, HID), jnp.float32)]
        ),
    )(idx_i, idx_j, pa, pb, rbfp_l, zeros128)


_EB = 1000


def _rbf_body(sq_ref, w1c_ref, *out_ref):
    d2 = jnp.sum(sq_ref[...], axis=1, keepdims=True)
    dist = jnp.sqrt(d2)
    centers = lax.broadcasted_iota(jnp.int32, (1, RBF), 1).astype(
        jnp.float32) * (CUTOFF / (RBF - 1))
    dlt = dist - centers
    rbf = jnp.exp(-GAMMA * dlt * dlt)
    res = jnp.dot(rbf, w1c_ref[...], preferred_element_type=jnp.float32)
    for l in range(NL):
        out_ref[l][...] = res[:, l * HID:(l + 1) * HID]


def _k_rbf(sq, w1c):
    return pl.pallas_call(
        _rbf_body,
        grid=(E // _EB,),
        in_specs=[
            pl.BlockSpec((_EB, 16), lambda i: (i, 0)),
            pl.BlockSpec((RBF, NL * HID), lambda i: (0, 0)),
        ],
        out_specs=[pl.BlockSpec((_EB, HID), lambda i: (i, 0))] * NL,
        out_shape=[jax.ShapeDtypeStruct((E, HID), jnp.float32)] * NL,
    )(sq, w1c)


_NB = 256


def _prep_body(h_ref, degp_ref, w1a_ref, w1b_ref, b1_ref,
               pa_ref, pb_ref, deg_ref, inv_ref):
    h = h_ref[...]
    dp = degp_ref[...]
    deg16 = dp[0] + dp[1]
    degc = deg16[:, 0:1]
    pa_ref[...] = jnp.dot(h, w1a_ref[...],
                          preferred_element_type=jnp.float32) + b1_ref[...]
    pb_ref[...] = jnp.dot(h, w1b_ref[...], preferred_element_type=jnp.float32)
    deg_ref[...] = degc
    inv_ref[...] = 1.0 / jnp.maximum(degc, 1.0)


def _k_prep(h0, degp, w1a0, w1b0, b1_0):
    o = jax.ShapeDtypeStruct((NPAD, HID), jnp.float32)
    oc = jax.ShapeDtypeStruct((NPAD, 1), jnp.float32)
    blk = pl.BlockSpec((_NB, HID), lambda i: (i, 0))
    col = pl.BlockSpec((_NB, 1), lambda i: (i, 0))
    return pl.pallas_call(
        _prep_body,
        grid=(NPAD // _NB,),
        in_specs=[
            blk,
            pl.BlockSpec((NC, _NB, 16), lambda i: (0, i, 0)),
            pl.BlockSpec((HID, HID), lambda i: (0, 0)),
            pl.BlockSpec((HID, HID), lambda i: (0, 0)),
            pl.BlockSpec((1, HID), lambda i: (0, 0)),
        ],
        out_specs=[blk, blk, col, col],
        out_shape=[o, o, oc, oc],
    )(h0, degp, w1a0, w1b0, b1_0)


def _post_body(want_next, accp_ref, deg_ref, inv_ref, h_ref,
               w2_ref, b2_ref, u1a_ref, u1b_ref, ub1_ref, u2_ref, ub2_ref,
               *rest):
    if want_next:
        wna_ref, wnb_ref, nb1_ref, hn_ref, pa_ref, pb_ref = rest
    else:
        (hn_ref,) = rest
    ap = accp_ref[...]
    acc = ap[0] + ap[1]
    aggr = (jnp.dot(acc, w2_ref[...], preferred_element_type=jnp.float32)
            + deg_ref[...] * b2_ref[...]) * inv_ref[...]
    h = h_ref[...]
    t = jnp.maximum(
        jnp.dot(h, u1a_ref[...], preferred_element_type=jnp.float32)
        + jnp.dot(aggr, u1b_ref[...], preferred_element_type=jnp.float32)
        + ub1_ref[...], 0.0)
    hn = jnp.dot(t, u2_ref[...], preferred_element_type=jnp.float32) + ub2_ref[...]
    hn_ref[...] = hn
    if want_next:
        pa_ref[...] = jnp.dot(hn, wna_ref[...],
                              preferred_element_type=jnp.float32) + nb1_ref[...]
        pb_ref[...] = jnp.dot(hn, wnb_ref[...],
                              preferred_element_type=jnp.float32)


def _k_post(accp, deg_b, inv_b, h, w2, b2, u1a, u1b, ub1, u2, ub2,
            nxt=None):
    want_next = nxt is not None
    o = jax.ShapeDtypeStruct((NPAD, HID), jnp.float32)
    full = lambda *shape: pl.BlockSpec(shape, lambda i: (0,) * len(shape))
    blk = pl.BlockSpec((_NB, HID), lambda i: (i, 0))
    col = pl.BlockSpec((_NB, 1), lambda i: (i, 0))
    in_specs = [
        pl.BlockSpec((NC, _NB, HID), lambda i: (0, i, 0)),
        col, col, blk,
        full(HID, HID), full(1, HID), full(HID, HID), full(HID, HID),
        full(1, HID), full(HID, HID), full(1, HID),
    ]
    args = [accp, deg_b, inv_b, h, w2, b2, u1a, u1b, ub1, u2, ub2]
    if want_next:
        in_specs += [full(HID, HID), full(HID, HID), full(1, HID)]
        args += list(nxt)
        out_specs, out_shape = [blk, blk, blk], [o, o, o]
    else:
        out_specs, out_shape = [blk], [o]
    return pl.pallas_call(
        functools.partial(_post_body, want_next),
        grid=(NPAD // _NB,),
        in_specs=in_specs,
        out_specs=out_specs,
        out_shape=out_shape,
    )(*args)


def _final_body(h_ref, ow1_ref, ob1_ref, ow2_ref, ob2_ref, out_ref):
    i = pl.program_id(0)
    h = h_ref[...]
    t = jnp.maximum(
        jnp.dot(h, ow1_ref[...], preferred_element_type=jnp.float32)
        + ob1_ref[...], 0.0)
    e = jnp.dot(t, ow2_ref[...], preferred_element_type=jnp.float32)
    row = lax.broadcasted_iota(jnp.int32, (_NB, 1), 0) + i * _NB
    s = jnp.sum(jnp.where(row < N, e, 0.0))
    sv = jnp.full((1, HID), s, jnp.float32)

    @pl.when(i == 0)
    def _():
        out_ref[...] = sv + N * ob2_ref[...]

    @pl.when(i != 0)
    def _():
        out_ref[...] += sv


def _k_final(h, ow1, ob1, ow2, ob2):
    return pl.pallas_call(
        _final_body,
        grid=(NPAD // _NB,),
        in_specs=[
            pl.BlockSpec((_NB, HID), lambda i: (i, 0)),
            pl.BlockSpec((HID, HID), lambda i: (0, 0)),
            pl.BlockSpec((1, HID), lambda i: (0, 0)),
            pl.BlockSpec((HID, 1), lambda i: (0, 0)),
            pl.BlockSpec((1, HID), lambda i: (0, 0)),
        ],
        out_specs=pl.BlockSpec((1, HID), lambda i: (0, 0)),
        out_shape=jax.ShapeDtypeStruct((1, HID), jnp.float32),
    )(h, ow1, ob1, ow2, ob2)


def kernel(z, pos, edge_index, emb, msg_w1, msg_b1, msg_w2, msg_b2,
           upd_w1, upd_b1, upd_w2, upd_b2, out_w1, out_b1, out_w2, out_b2):
    idx_i = edge_index[0].astype(jnp.int32)
    idx_j = edge_index[1].astype(jnp.int32)
    posp = jnp.pad(pos.astype(jnp.float32), ((0, 0), (0, 13)))

    e1_rows = jnp.zeros((CHG, 16), jnp.float32).at[:, 0].set(1.0)
    zeros16 = jnp.zeros((NPAD, 16), jnp.float32)
    zeros128 = jnp.zeros((NPAD, HID), jnp.float32)

    z_pad = jnp.zeros((NPAD,), jnp.int32).at[:N].set(z.astype(jnp.int32))
    sq, degp, h0 = _sc_geom(idx_i, idx_j, posp, z_pad, emb, e1_rows, zeros16)

    w1c = jnp.concatenate([msg_w1[l, 2 * HID:, :] for l in range(NL)],
                          axis=1)
    rbfp = _k_rbf(sq, w1c)

    b1 = msg_b1.reshape(NL, 1, HID)
    b2 = msg_b2.reshape(NL, 1, HID)
    ub1 = upd_b1.reshape(NL, 1, HID)
    ub2 = upd_b2.reshape(NL, 1, HID)

    pa, pb, deg_b, inv_b = _k_prep(
        h0, degp, msg_w1[0, :HID, :], msg_w1[0, HID:2 * HID, :], b1[0])
    h = h0

    for l in range(NL):
        accp = _sc_edge(idx_i, idx_j, pa, pb, rbfp[l], zeros128)
        if l < NL - 1:
            nxt = (msg_w1[l + 1, :HID, :], msg_w1[l + 1, HID:2 * HID, :],
                   b1[l + 1])
            h, pa, pb = _k_post(
                accp, deg_b, inv_b, h, msg_w2[l], b2[l],
                upd_w1[l, :HID, :], upd_w1[l, HID:, :], ub1[l],
                upd_w2[l], ub2[l], nxt=nxt)
        else:
            (h,) = _k_post(
                accp, deg_b, inv_b, h, msg_w2[l], b2[l],
                upd_w1[l, :HID, :], upd_w1[l, HID:, :], ub1[l],
                upd_w2[l], ub2[l])

    ob2_row = jnp.broadcast_to(out_b2.reshape(1, 1), (1, HID))
    res = _k_final(h, out_w1, out_b1.reshape(1, HID), out_w2, ob2_row)
    return res[0, 0]
